# baseline (speedup 1.0000x reference)
import functools

import jax
import jax.numpy as jnp
from jax import lax
from jax.experimental import pallas as pl
from jax.experimental.pallas import tpu as pltpu
from jax.experimental.pallas import tpu_sc as plsc

NC = 2
NS = 16
NW = NC * NS
LANES = 16

CHUNK = 40
VC = 512


def _mesh():
  return plsc.VectorSubcoreMesh(
      core_axis_name="c", subcore_axis_name="s", num_cores=NC,
      num_subcores=NS)


def _sc_transpose(tab_t, tail_flat, V, D):
  n_full = V // VC
  rem = V - n_full * VC
  per_w = (n_full + NW - 1) // NW
  DW = D // 2

  @functools.partial(
      pl.kernel,
      out_type=jax.ShapeDtypeStruct((V * DW,), jnp.int32),
      mesh=_mesh(),
      compiler_params=pltpu.CompilerParams(needs_layout_passes=False),
      scratch_types=[
          pltpu.VMEM((D, VC), jnp.float32),
          pltpu.VMEM((D, VC), jnp.float32),
          pltpu.VMEM((VC * DW,), jnp.int32),
          pltpu.VMEM((VC * DW,), jnp.int32),
          pltpu.SemaphoreType.DMA,
          pltpu.SemaphoreType.DMA,
          pltpu.SemaphoreType.DMA,
          pltpu.SemaphoreType.DMA,
      ],
  )
  def tr_kernel(tab_hbm, tail_hbm, out_hbm, in_a, in_b, out_a, out_b,
                isem_a, isem_b, osem_a, osem_b):
    c = lax.axis_index("c")
    s = lax.axis_index("s")
    wid = s * NC + c
    iota = lax.iota(jnp.int32, LANES)

    def fire_in(t, in_v, isem):
      v0 = (wid + t * NW) * VC
      pltpu.async_copy(tab_hbm.at[pl.ds(0, D), pl.ds(v0, VC)], in_v, isem)

    def wait_in(in_v, isem):
      pltpu.make_async_copy(
          tab_hbm.at[pl.ds(0, D), pl.ds(0, VC)], in_v, isem).wait()

    rots = [(iota + d) & (LANES - 1) for d in range(LANES)]
    lo_rows = iota
    hi_rows = iota + LANES
    pk_base = [rots[d] * DW + iota for d in range(LANES)]

    def transpose(in_v, out_v):
      @pl.loop(0, VC // LANES)
      def _(g):
        c0 = g * LANES
        vbase = c0 * DW
        for d in range(LANES):
          cidx = rots[d] + c0
          lo = plsc.load_gather(in_v, [lo_rows, cidx])
          hi = plsc.load_gather(in_v, [hi_rows, cidx])
          u = plsc.bitcast(
              plsc.pack(lo, hi, format=plsc.PackFormat.INTERLEAVED),
              jnp.int32)
          plsc.store_scatter(out_v, [pk_base[d] + vbase], u)

    def fire_out(t, out_v, osem):
      v0 = (wid + t * NW) * VC
      pltpu.async_copy(out_v, out_hbm.at[pl.ds(v0 * DW, VC * DW)], osem)

    def wait_out(out_v, osem):
      pltpu.make_async_copy(
          out_v, out_hbm.at[pl.ds(0, VC * DW)], osem).wait()

    def work(t):
      return wid + t * NW < n_full

    @pl.when(work(0))
    def _():
      fire_in(0, in_a, isem_a)

    @pl.loop(0, per_w, step=2)
    def _(t):
      @pl.when(work(t + 1))
      def _():
        fire_in(t + 1, in_b, isem_b)

      @pl.when(work(t))
      def _():
        wait_in(in_a, isem_a)

        @pl.when(t >= 2)
        def _():
          wait_out(out_a, osem_a)
        transpose(in_a, out_a)
        fire_out(t, out_a, osem_a)

        @pl.when(work(t + 2))
        def _():
          fire_in(t + 2, in_a, isem_a)

      @pl.when(work(t + 1))
      def _():
        wait_in(in_b, isem_b)

        @pl.when(t >= 2)
        def _():
          wait_out(out_b, osem_b)
        transpose(in_b, out_b)
        fire_out(t + 1, out_b, osem_b)

    @pl.when(work(0))
    def _():
      wait_out(out_a, osem_a)

    @pl.when(work(1))
    def _():
      wait_out(out_b, osem_b)

    if rem:
      @pl.when(wid == 0)
      def _():
        pltpu.sync_copy(
            tail_hbm, out_hbm.at[pl.ds(n_full * VC * DW, rem * DW)])

  return tr_kernel(tab_t, tail_flat)


def _sc_pooled_sums(x, tab_lin, B, L, D, V):
  CB = B // NW
  SPB = L // CHUNK
  x1 = x.reshape(-1)
  tab2 = tab_lin.reshape(V, D)

  @functools.partial(
      pl.kernel,
      out_type=jax.ShapeDtypeStruct((B, D), jnp.float32),
      mesh=_mesh(),
      compiler_params=pltpu.CompilerParams(
          use_tc_tiling_on_sc=False, needs_layout_passes=False),
      scratch_types=[
          pltpu.VMEM((CB * L,), jnp.int32),
          pltpu.VMEM((L, D), jnp.bfloat16),
          pltpu.VMEM((L, D), jnp.bfloat16),
          pltpu.VMEM((CB, D), jnp.float32),
          pltpu.SemaphoreType.DMA,
          pltpu.SemaphoreType.DMA,
      ],
  )
  def sc_kernel(x_hbm, tab_hbm, out_hbm, idx_v, rbuf_a, rbuf_b, sum_v,
                sem_a, sem_b):
    c = lax.axis_index("c")
    s = lax.axis_index("s")
    wid = s * NC + c

    pltpu.sync_copy(x_hbm.at[pl.ds(wid * CB * L, CB * L)], idx_v)

    def fire(i, rbuf, sem):
      for j in range(SPB):
        pltpu.async_copy(
            tab_hbm.at[idx_v.at[pl.ds(i * L + j * CHUNK, CHUNK)]],
            rbuf.at[pl.ds(j * CHUNK, CHUNK)],
            sem)

    def wait(rbuf, sem):
      pltpu.make_async_copy(tab_hbm.at[pl.ds(0, L)], rbuf, sem).wait()

    def accumulate(i, rbuf):
      accs = [jnp.zeros((LANES,), jnp.float32) for _ in range(8)]
      for k in range(L):
        p = k % 4
        u = plsc.bitcast(rbuf[k, pl.ds(0, D)], jnp.uint32)
        lo = plsc.bitcast(u << jnp.uint32(16), jnp.float32)
        hi = plsc.bitcast(u & jnp.uint32(0xFFFF0000), jnp.float32)
        accs[2 * p] = accs[2 * p] + lo
        accs[2 * p + 1] = accs[2 * p + 1] + hi
      lo = (accs[0] + accs[2]) + (accs[4] + accs[6])
      hi = (accs[1] + accs[3]) + (accs[5] + accs[7])
      sum_v[i, pl.ds(0, LANES)] = lo
      sum_v[i, pl.ds(LANES, LANES)] = hi

    fire(0, rbuf_a, sem_a)

    @pl.loop(0, CB, step=2)
    def _(i):
      fire(i + 1, rbuf_b, sem_b)
      wait(rbuf_a, sem_a)
      accumulate(i, rbuf_a)

      @pl.when(i + 2 < CB)
      def _():
        fire(i + 2, rbuf_a, sem_a)

      wait(rbuf_b, sem_b)
      accumulate(i + 1, rbuf_b)

    pltpu.sync_copy(sum_v, out_hbm.at[pl.ds(wid * CB, CB)])

  return sc_kernel(x1, tab2)


def _tc_linear(sums, W, b2, L):
  B, D = sums.shape
  NCLS = W.shape[1]

  def body(s_ref, w_ref, b_ref, o_ref):
    m = s_ref[...] * jnp.float32(1.0 / L)
    o_ref[...] = (
        jnp.dot(m, w_ref[...], preferred_element_type=jnp.float32)
        + b_ref[...])

  return pl.pallas_call(
      body,
      out_shape=jax.ShapeDtypeStruct((B, NCLS), jnp.float32),
  )(sums, W, b2)


def kernel(x, emb_table, W, b):
  B, L = x.shape
  V, D = emb_table.shape
  x = x.astype(jnp.int32)
  n_tail = V % VC
  t = emb_table[V - n_tail:, :]
  tb = jnp.stack([t[:, :D // 2], t[:, D // 2:]], axis=-1).astype(
      jnp.bfloat16)
  tail_flat = jax.lax.bitcast_convert_type(tb, jnp.int32).reshape(-1)
  tab_i32 = _sc_transpose(emb_table.T, tail_flat, V, D)
  tab_bf = jax.lax.bitcast_convert_type(
      tab_i32, jnp.bfloat16).reshape(V, D)
  sums = _sc_pooled_sums(x, tab_bf, B, L, D, V)
  return _tc_linear(sums, W, b.reshape(1, -1).astype(jnp.float32), L)


# speedup vs baseline: 5.7010x; 5.7010x over previous
import functools

import jax
import jax.numpy as jnp
from jax import lax
from jax.experimental import pallas as pl
from jax.experimental.pallas import tpu as pltpu
from jax.experimental.pallas import tpu_sc as plsc

NC = 2
NS = 16
NW = NC * NS
LANES = 16

CHUNK = 40
VC = 512


def _mesh():
  return plsc.VectorSubcoreMesh(
      core_axis_name="c", subcore_axis_name="s", num_cores=NC,
      num_subcores=NS)


def _sc_transpose(tab_t, tail_flat, V, D):
  n_full = V // VC
  rem = V - n_full * VC
  per_w = (n_full + NW - 1) // NW
  DW = D // 2

  @functools.partial(
      pl.kernel,
      out_type=jax.ShapeDtypeStruct((V * DW,), jnp.int32),
      mesh=_mesh(),
      compiler_params=pltpu.CompilerParams(needs_layout_passes=False),
      scratch_types=[
          pltpu.VMEM((D, VC), jnp.float32),
          pltpu.VMEM((D, VC), jnp.float32),
          pltpu.VMEM((VC * DW,), jnp.int32),
          pltpu.VMEM((VC * DW,), jnp.int32),
          pltpu.SemaphoreType.DMA,
          pltpu.SemaphoreType.DMA,
          pltpu.SemaphoreType.DMA,
          pltpu.SemaphoreType.DMA,
      ],
  )
  def tr_kernel(tab_hbm, tail_hbm, out_hbm, in_a, in_b, out_a, out_b,
                isem_a, isem_b, osem_a, osem_b):
    c = lax.axis_index("c")
    s = lax.axis_index("s")
    wid = s * NC + c
    iota = lax.iota(jnp.int32, LANES)

    def fire_in(t, in_v, isem):
      v0 = (wid + t * NW) * VC
      pltpu.async_copy(tab_hbm.at[pl.ds(0, D), pl.ds(v0, VC)], in_v, isem)

    def wait_in(in_v, isem):
      pltpu.make_async_copy(
          tab_hbm.at[pl.ds(0, D), pl.ds(0, VC)], in_v, isem).wait()

    rots = [(iota + d) & (LANES - 1) for d in range(LANES)]
    lo_rows = iota
    hi_rows = iota + LANES
    pk_base = [rots[d] * DW + iota for d in range(LANES)]

    def transpose(in_v, out_v):
      @pl.loop(0, VC // LANES)
      def _(g):
        c0 = g * LANES
        vbase = c0 * DW
        for d in range(LANES):
          cidx = rots[d] + c0
          lo = plsc.load_gather(in_v, [lo_rows, cidx])
          hi = plsc.load_gather(in_v, [hi_rows, cidx])
          u = plsc.bitcast(
              plsc.pack(lo, hi, format=plsc.PackFormat.INTERLEAVED),
              jnp.int32)
          plsc.store_scatter(out_v, [pk_base[d] + vbase], u)

    def fire_out(t, out_v, osem):
      v0 = (wid + t * NW) * VC
      pltpu.async_copy(out_v, out_hbm.at[pl.ds(v0 * DW, VC * DW)], osem)

    def wait_out(out_v, osem):
      pltpu.make_async_copy(
          out_v, out_hbm.at[pl.ds(0, VC * DW)], osem).wait()

    def work(t):
      return wid + t * NW < n_full

    @pl.when(work(0))
    def _():
      fire_in(0, in_a, isem_a)

    @pl.loop(0, per_w, step=2)
    def _(t):
      @pl.when(work(t + 1))
      def _():
        fire_in(t + 1, in_b, isem_b)

      @pl.when(work(t))
      def _():
        wait_in(in_a, isem_a)

        @pl.when(t >= 2)
        def _():
          wait_out(out_a, osem_a)
        transpose(in_a, out_a)
        fire_out(t, out_a, osem_a)

        @pl.when(work(t + 2))
        def _():
          fire_in(t + 2, in_a, isem_a)

      @pl.when(work(t + 1))
      def _():
        wait_in(in_b, isem_b)

        @pl.when(t >= 2)
        def _():
          wait_out(out_b, osem_b)
        transpose(in_b, out_b)
        fire_out(t + 1, out_b, osem_b)

    @pl.when(work(0))
    def _():
      wait_out(out_a, osem_a)

    @pl.when(work(1))
    def _():
      wait_out(out_b, osem_b)

    if rem:
      @pl.when(wid == 0)
      def _():
        pltpu.sync_copy(
            tail_hbm, out_hbm.at[pl.ds(n_full * VC * DW, rem * DW)])

  return tr_kernel(tab_t, tail_flat)


def _sc_pooled_sums(x, tab_i32, B, L, D, V):
  CB = B // NW
  SPB = L // CHUNK
  DW = D // 2
  x1 = x.reshape(-1)
  tab2 = tab_i32.reshape(V, DW)

  @functools.partial(
      pl.kernel,
      out_type=jax.ShapeDtypeStruct((B, D), jnp.float32),
      mesh=_mesh(),
      compiler_params=pltpu.CompilerParams(
          use_tc_tiling_on_sc=False, needs_layout_passes=False),
      scratch_types=[
          pltpu.VMEM((CB * L,), jnp.int32),
          pltpu.VMEM((L, DW), jnp.int32),
          pltpu.VMEM((L, DW), jnp.int32),
          pltpu.VMEM((CB, D), jnp.float32),
          pltpu.SemaphoreType.DMA,
          pltpu.SemaphoreType.DMA,
      ],
  )
  def sc_kernel(x_hbm, tab_hbm, out_hbm, idx_v, rbuf_a, rbuf_b, sum_v,
                sem_a, sem_b):
    c = lax.axis_index("c")
    s = lax.axis_index("s")
    wid = s * NC + c

    pltpu.sync_copy(x_hbm.at[pl.ds(wid * CB * L, CB * L)], idx_v)

    def fire(i, rbuf, sem):
      for j in range(SPB):
        pltpu.async_copy(
            tab_hbm.at[idx_v.at[pl.ds(i * L + j * CHUNK, CHUNK)]],
            rbuf.at[pl.ds(j * CHUNK, CHUNK)],
            sem)

    def wait(rbuf, sem):
      pltpu.make_async_copy(tab_hbm.at[pl.ds(0, L)], rbuf, sem).wait()

    def accumulate(i, rbuf):
      accs = [jnp.zeros((LANES,), jnp.float32) for _ in range(8)]
      for k in range(L):
        p = k % 4
        u = plsc.bitcast(rbuf[k, pl.ds(0, DW)], jnp.uint32)
        lo = plsc.bitcast(u << jnp.uint32(16), jnp.float32)
        hi = plsc.bitcast(u & jnp.uint32(0xFFFF0000), jnp.float32)
        accs[2 * p] = accs[2 * p] + lo
        accs[2 * p + 1] = accs[2 * p + 1] + hi
      lo = (accs[0] + accs[2]) + (accs[4] + accs[6])
      hi = (accs[1] + accs[3]) + (accs[5] + accs[7])
      sum_v[i, pl.ds(0, LANES)] = lo
      sum_v[i, pl.ds(LANES, LANES)] = hi

    fire(0, rbuf_a, sem_a)

    @pl.loop(0, CB, step=2)
    def _(i):
      fire(i + 1, rbuf_b, sem_b)
      wait(rbuf_a, sem_a)
      accumulate(i, rbuf_a)

      @pl.when(i + 2 < CB)
      def _():
        fire(i + 2, rbuf_a, sem_a)

      wait(rbuf_b, sem_b)
      accumulate(i + 1, rbuf_b)

    pltpu.sync_copy(sum_v, out_hbm.at[pl.ds(wid * CB, CB)])

  return sc_kernel(x1, tab2)


def _tc_linear(sums, W, b2, L):
  B, D = sums.shape
  NCLS = W.shape[1]

  def body(s_ref, w_ref, b_ref, o_ref):
    m = s_ref[...] * jnp.float32(1.0 / L)
    o_ref[...] = (
        jnp.dot(m, w_ref[...], preferred_element_type=jnp.float32)
        + b_ref[...])

  return pl.pallas_call(
      body,
      out_shape=jax.ShapeDtypeStruct((B, NCLS), jnp.float32),
  )(sums, W, b2)


def kernel(x, emb_table, W, b):
  B, L = x.shape
  V, D = emb_table.shape
  x = x.astype(jnp.int32)
  n_tail = V % VC
  t = emb_table[V - n_tail:, :]
  tb = jnp.stack([t[:, :D // 2], t[:, D // 2:]], axis=-1).astype(
      jnp.bfloat16)
  tail_flat = jax.lax.bitcast_convert_type(tb, jnp.int32).reshape(-1)
  tab_i32 = _sc_transpose(emb_table.T, tail_flat, V, D)
  sums = _sc_pooled_sums(x, tab_i32, B, L, D, V)
  return _tc_linear(sums, W, b.reshape(1, -1).astype(jnp.float32), L)


# speedup vs baseline: 6.9209x; 1.2140x over previous
import functools

import jax
import jax.numpy as jnp
from jax import lax
from jax.experimental import pallas as pl
from jax.experimental.pallas import tpu as pltpu
from jax.experimental.pallas import tpu_sc as plsc

NC = 2
NS = 16
NW = NC * NS
LANES = 16

CHUNK = 40
VC = 512


def _mesh():
  return plsc.VectorSubcoreMesh(
      core_axis_name="c", subcore_axis_name="s", num_cores=NC,
      num_subcores=NS)


def _sc_transpose(tab_t, tail_flat, V, D):
  n_full = V // VC
  rem = V - n_full * VC
  per_w = (n_full + NW - 1) // NW
  DW = D // 2

  @functools.partial(
      pl.kernel,
      out_type=jax.ShapeDtypeStruct((V * DW,), jnp.int32),
      mesh=_mesh(),
      compiler_params=pltpu.CompilerParams(needs_layout_passes=False),
      scratch_types=[
          pltpu.VMEM((D, VC), jnp.float32),
          pltpu.VMEM((D, VC), jnp.float32),
          pltpu.VMEM((VC * DW,), jnp.int32),
          pltpu.VMEM((VC * DW,), jnp.int32),
          pltpu.VMEM((max(rem, 1) * D,), jnp.float32),
          pltpu.SemaphoreType.DMA,
          pltpu.SemaphoreType.DMA,
          pltpu.SemaphoreType.DMA,
          pltpu.SemaphoreType.DMA,
      ],
  )
  def tr_kernel(tab_hbm, tail_hbm, out_hbm, in_a, in_b, out_a, out_b,
                tail_v, isem_a, isem_b, osem_a, osem_b):
    c = lax.axis_index("c")
    s = lax.axis_index("s")
    wid = s * NC + c
    iota = lax.iota(jnp.int32, LANES)

    def fire_in(t, in_v, isem):
      v0 = (wid + t * NW) * VC
      pltpu.async_copy(tab_hbm.at[pl.ds(0, D), pl.ds(v0, VC)], in_v, isem)

    def wait_in(in_v, isem):
      pltpu.make_async_copy(
          tab_hbm.at[pl.ds(0, D), pl.ds(0, VC)], in_v, isem).wait()

    rots = [(iota + d) & (LANES - 1) for d in range(LANES)]
    lo_rows = iota
    hi_rows = iota + LANES
    pk_base = [rots[d] * DW + iota for d in range(LANES)]

    def transpose(in_v, out_v):
      @pl.loop(0, VC // LANES)
      def _(g):
        c0 = g * LANES
        vbase = c0 * DW
        for d in range(LANES):
          cidx = rots[d] + c0
          lo = plsc.load_gather(in_v, [lo_rows, cidx])
          hi = plsc.load_gather(in_v, [hi_rows, cidx])
          u = plsc.bitcast(
              plsc.pack(lo, hi, format=plsc.PackFormat.INTERLEAVED),
              jnp.int32)
          plsc.store_scatter(out_v, [pk_base[d] + vbase], u)

    def fire_out(t, out_v, osem):
      v0 = (wid + t * NW) * VC
      pltpu.async_copy(out_v, out_hbm.at[pl.ds(v0 * DW, VC * DW)], osem)

    def wait_out(out_v, osem):
      pltpu.make_async_copy(
          out_v, out_hbm.at[pl.ds(0, VC * DW)], osem).wait()

    def work(t):
      return wid + t * NW < n_full

    @pl.when(work(0))
    def _():
      fire_in(0, in_a, isem_a)

    @pl.loop(0, per_w, step=2)
    def _(t):
      @pl.when(work(t + 1))
      def _():
        fire_in(t + 1, in_b, isem_b)

      @pl.when(work(t))
      def _():
        wait_in(in_a, isem_a)

        @pl.when(t >= 2)
        def _():
          wait_out(out_a, osem_a)
        transpose(in_a, out_a)
        fire_out(t, out_a, osem_a)

        @pl.when(work(t + 2))
        def _():
          fire_in(t + 2, in_a, isem_a)

      @pl.when(work(t + 1))
      def _():
        wait_in(in_b, isem_b)

        @pl.when(t >= 2)
        def _():
          wait_out(out_b, osem_b)
        transpose(in_b, out_b)
        fire_out(t + 1, out_b, osem_b)

    @pl.when(work(0))
    def _():
      wait_out(out_a, osem_a)

    @pl.when(work(1))
    def _():
      wait_out(out_b, osem_b)

    if rem:
      @pl.when(wid == 0)
      def _():
        pltpu.sync_copy(tail_hbm, tail_v)
        for v in range(rem):
          lo = tail_v[pl.ds(v * D, LANES)]
          hi = tail_v[pl.ds(v * D + LANES, LANES)]
          u = plsc.bitcast(
              plsc.pack(lo, hi, format=plsc.PackFormat.INTERLEAVED),
              jnp.int32)
          out_a[pl.ds(v * DW, LANES)] = u
        pltpu.sync_copy(
            out_a.at[pl.ds(0, rem * DW)],
            out_hbm.at[pl.ds(n_full * VC * DW, rem * DW)])

  return tr_kernel(tab_t, tail_flat)


def _sc_pooled_sums(x, tab_i32, B, L, D, V):
  CB = B // NW
  SPB = L // CHUNK
  DW = D // 2
  x1 = x.reshape(-1)
  tab2 = tab_i32.reshape(V, DW)

  @functools.partial(
      pl.kernel,
      out_type=jax.ShapeDtypeStruct((B, D), jnp.float32),
      mesh=_mesh(),
      compiler_params=pltpu.CompilerParams(
          use_tc_tiling_on_sc=False, needs_layout_passes=False),
      scratch_types=[
          pltpu.VMEM((CB * L,), jnp.int32),
          pltpu.VMEM((L, DW), jnp.int32),
          pltpu.VMEM((L, DW), jnp.int32),
          pltpu.VMEM((CB, D), jnp.float32),
          pltpu.SemaphoreType.DMA,
          pltpu.SemaphoreType.DMA,
      ],
  )
  def sc_kernel(x_hbm, tab_hbm, out_hbm, idx_v, rbuf_a, rbuf_b, sum_v,
                sem_a, sem_b):
    c = lax.axis_index("c")
    s = lax.axis_index("s")
    wid = s * NC + c

    pltpu.sync_copy(x_hbm.at[pl.ds(wid * CB * L, CB * L)], idx_v)

    def fire(i, rbuf, sem):
      for j in range(SPB):
        pltpu.async_copy(
            tab_hbm.at[idx_v.at[pl.ds(i * L + j * CHUNK, CHUNK)]],
            rbuf.at[pl.ds(j * CHUNK, CHUNK)],
            sem)

    def wait(rbuf, sem):
      pltpu.make_async_copy(tab_hbm.at[pl.ds(0, L)], rbuf, sem).wait()

    def accumulate(i, rbuf):
      accs = [jnp.zeros((LANES,), jnp.float32) for _ in range(8)]
      for k in range(L):
        p = k % 4
        u = plsc.bitcast(rbuf[k, pl.ds(0, DW)], jnp.uint32)
        lo = plsc.bitcast(u << jnp.uint32(16), jnp.float32)
        hi = plsc.bitcast(u & jnp.uint32(0xFFFF0000), jnp.float32)
        accs[2 * p] = accs[2 * p] + lo
        accs[2 * p + 1] = accs[2 * p + 1] + hi
      lo = (accs[0] + accs[2]) + (accs[4] + accs[6])
      hi = (accs[1] + accs[3]) + (accs[5] + accs[7])
      sum_v[i, pl.ds(0, LANES)] = lo
      sum_v[i, pl.ds(LANES, LANES)] = hi

    fire(0, rbuf_a, sem_a)

    @pl.loop(0, CB, step=2)
    def _(i):
      fire(i + 1, rbuf_b, sem_b)
      wait(rbuf_a, sem_a)
      accumulate(i, rbuf_a)

      @pl.when(i + 2 < CB)
      def _():
        fire(i + 2, rbuf_a, sem_a)

      wait(rbuf_b, sem_b)
      accumulate(i + 1, rbuf_b)

    pltpu.sync_copy(sum_v, out_hbm.at[pl.ds(wid * CB, CB)])

  return sc_kernel(x1, tab2)


def _tc_linear(sums, W, b2, L):
  B, D = sums.shape
  NCLS = W.shape[1]

  def body(s_ref, w_ref, b_ref, o_ref):
    m = s_ref[...] * jnp.float32(1.0 / L)
    o_ref[...] = (
        jnp.dot(m, w_ref[...], preferred_element_type=jnp.float32)
        + b_ref[...])

  return pl.pallas_call(
      body,
      out_shape=jax.ShapeDtypeStruct((B, NCLS), jnp.float32),
  )(sums, W, b2)


def kernel(x, emb_table, W, b):
  B, L = x.shape
  V, D = emb_table.shape
  x = x.astype(jnp.int32)
  n_tail = V % VC
  tail_flat = emb_table[V - n_tail:, :].reshape(-1)
  tab_i32 = _sc_transpose(emb_table.T, tail_flat, V, D)
  sums = _sc_pooled_sums(x, tab_i32, B, L, D, V)
  return _tc_linear(sums, W, b.reshape(1, -1).astype(jnp.float32), L)
